# baseline (speedup 1.0000x reference)
import jax
import jax.numpy as jnp
from jax.experimental import pallas as pl
from jax.experimental.pallas import tpu as pltpu

SEQ_LEN = 8192
D = 768
B = 2
EPS = 1e-12

ROWS = 1024
SPLIT = 12288


def _ln_body(x_ref, pe_ref, o_ref):
    emb = x_ref[...] + pe_ref[...]
    mean = jnp.mean(emb, axis=1, keepdims=True)
    c = emb - mean
    var = jnp.mean(c * c, axis=1, keepdims=True)
    o_ref[...] = c * jax.lax.rsqrt(var + EPS)


def _part(xf, pos_emb, row0, nrows):
    nblk = nrows // ROWS
    blk0 = row0 // ROWS
    pe_blocks = SEQ_LEN // ROWS
    return pl.pallas_call(
        _ln_body,
        grid=(nblk,),
        in_specs=[
            pl.BlockSpec((ROWS, D), lambda i: (i + blk0, 0)),
            pl.BlockSpec((ROWS, D), lambda i: ((i + blk0) % pe_blocks, 0)),
        ],
        out_specs=pl.BlockSpec((ROWS, D), lambda i: (i, 0)),
        out_shape=jax.ShapeDtypeStruct((nrows, D), xf.dtype),
    )(xf, pos_emb)


@jax.jit
def kernel(x, pos_emb, ln_w, ln_b):
    xf = x.reshape(B * SEQ_LEN, D)
    a = _part(xf, pos_emb, 0, SPLIT)
    b = _part(xf, pos_emb, SPLIT, B * SEQ_LEN - SPLIT)
    return jnp.concatenate([a, b], axis=0).reshape(B, SEQ_LEN, D)


# speedup vs baseline: 1.7988x; 1.7988x over previous
import jax
import jax.numpy as jnp
from jax.experimental import pallas as pl

SEQ_LEN = 8192
D = 768
B = 2
EPS = 1e-12

ROWS = 1024


def _ln_body(x_ref, pe_ref, w_ref, b_ref, o_ref):
    emb = (x_ref[...] + pe_ref[None]).reshape(B * ROWS, D)
    ones = jnp.ones((D, 1), jnp.float32)
    dn = (((1,), (0,)), ((), ()))
    s1 = jax.lax.dot_general(emb, ones, dn, preferred_element_type=jnp.float32)
    s2 = jax.lax.dot_general(emb * emb, ones, dn,
                             preferred_element_type=jnp.float32)
    mean = s1 * (1.0 / D)
    var = s2 * (1.0 / D) - mean * mean
    alpha = jax.lax.rsqrt(var + EPS)
    beta = -mean * alpha
    out = (emb * alpha + beta) * w_ref[...] + b_ref[...]
    o_ref[...] = out.reshape(B, ROWS, D)


@jax.jit
def kernel(x, pos_emb, ln_w, ln_b):
    w2 = ln_w.reshape(1, D)
    b2 = ln_b.reshape(1, D)
    return pl.pallas_call(
        _ln_body,
        grid=(SEQ_LEN // ROWS,),
        in_specs=[
            pl.BlockSpec((B, ROWS, D), lambda i: (0, i, 0)),
            pl.BlockSpec((ROWS, D), lambda i: (i, 0)),
            pl.BlockSpec((1, D), lambda i: (0, 0)),
            pl.BlockSpec((1, D), lambda i: (0, 0)),
        ],
        out_specs=pl.BlockSpec((B, ROWS, D), lambda i: (0, i, 0)),
        out_shape=jax.ShapeDtypeStruct((B, SEQ_LEN, D), x.dtype),
    )(x, pos_emb, w2, b2)


# speedup vs baseline: 1.8517x; 1.0294x over previous
import jax
import jax.numpy as jnp
from jax.experimental import pallas as pl

SEQ_LEN = 8192
D = 768
B = 2
EPS = 1e-12

ROWS = 1024


def _ln_body(x_ref, pe_ref, w_ref, b_ref, o_ref):
    emb = x_ref[...] + pe_ref[None]
    mean = jnp.mean(emb, axis=2, keepdims=True)
    c = emb - mean
    var = jnp.mean(c * c, axis=2, keepdims=True)
    o_ref[...] = c * jax.lax.rsqrt(var + EPS) * w_ref[...] + b_ref[...]


@jax.jit
def kernel(x, pos_emb, ln_w, ln_b):
    w2 = ln_w.reshape(1, D)
    b2 = ln_b.reshape(1, D)
    return pl.pallas_call(
        _ln_body,
        grid=(SEQ_LEN // ROWS,),
        in_specs=[
            pl.BlockSpec((B, ROWS, D), lambda i: (0, i, 0)),
            pl.BlockSpec((ROWS, D), lambda i: (i, 0)),
            pl.BlockSpec((1, D), lambda i: (0, 0)),
            pl.BlockSpec((1, D), lambda i: (0, 0)),
        ],
        out_specs=pl.BlockSpec((B, ROWS, D), lambda i: (0, i, 0)),
        out_shape=jax.ShapeDtypeStruct((B, SEQ_LEN, D), x.dtype),
    )(x, pos_emb, w2, b2)
